# baseline (speedup 1.0000x reference)
import functools

import jax
import jax.numpy as jnp
from jax import lax
from jax.experimental import pallas as pl
from jax.experimental.pallas import tpu as pltpu
from jax.experimental.pallas import tpu_sc as plsc

_BATCH = 4096
_SEQ = 200
_D = 64
_N = _BATCH * _SEQ
_NC, _NS = 2, 16
_NW = _NC * _NS
_ROWS_PER_W = _N // _NW
_CH = 128
_K = 8
_R = _CH * _K
_G = _ROWS_PER_W // _R

_mesh = plsc.VectorSubcoreMesh(core_axis_name="c", subcore_axis_name="s")


@functools.partial(
    pl.kernel,
    mesh=_mesh,
    out_type=jax.ShapeDtypeStruct((_N, _D), jnp.float32),
    scratch_types=[
        pltpu.VMEM((_R,), jnp.int32),
        pltpu.VMEM((_R, _D), jnp.float32),
        pltpu.SemaphoreType.DMA,
    ],
    compiler_params=pltpu.CompilerParams(use_tc_tiling_on_sc=False),
)
def _embed_gather(table_hbm, idx_hbm, out_hbm, idx_v, rows_v, sem):
    wid = lax.axis_index("s") * _NC + lax.axis_index("c")
    base = wid * _ROWS_PER_W

    def group(g, carry):
        row0 = base + g * _R
        pltpu.sync_copy(idx_hbm.at[pl.ds(row0, _R)], idx_v)
        handles = []
        for j in range(_K):
            handles.append(
                pltpu.async_copy(
                    table_hbm.at[idx_v.at[pl.ds(j * _CH, _CH)]],
                    rows_v.at[pl.ds(j * _CH, _CH)],
                    sem,
                )
            )
        for h in handles:
            h.wait()
        pltpu.sync_copy(rows_v, out_hbm.at[pl.ds(row0, _R)])
        return carry

    lax.fori_loop(0, _G, group, 0)


def kernel(token_ids, table):
    idx = token_ids.reshape(-1).astype(jnp.int32)
    out = _embed_gather(table, idx)
    return out.reshape(_BATCH, _SEQ, _D)


# speedup vs baseline: 1.0168x; 1.0168x over previous
import functools

import jax
import jax.numpy as jnp
from jax import lax
from jax.experimental import pallas as pl
from jax.experimental.pallas import tpu as pltpu
from jax.experimental.pallas import tpu_sc as plsc

_BATCH = 4096
_SEQ = 200
_D = 64
_N = _BATCH * _SEQ
_NC, _NS = 2, 16
_NW = _NC * _NS
_ROWS_PER_W = _N // _NW
_CH = 128
_K = 4
_R = _CH * _K
_G = _ROWS_PER_W // _R

_mesh = plsc.VectorSubcoreMesh(core_axis_name="c", subcore_axis_name="s")


@functools.partial(
    pl.kernel,
    mesh=_mesh,
    out_type=jax.ShapeDtypeStruct((_N, _D), jnp.float32),
    scratch_types=[
        pltpu.VMEM((_ROWS_PER_W,), jnp.int32),
        pltpu.VMEM((_R, _D), jnp.float32),
        pltpu.VMEM((_R, _D), jnp.float32),
        pltpu.SemaphoreType.DMA,
        pltpu.SemaphoreType.DMA,
    ],
    compiler_params=pltpu.CompilerParams(use_tc_tiling_on_sc=False),
)
def _embed_gather(table_hbm, idx_hbm, out_hbm, idx_v, rows0, rows1, sem0, sem1):
    wid = lax.axis_index("s") * _NC + lax.axis_index("c")
    base = wid * _ROWS_PER_W
    pltpu.sync_copy(idx_hbm.at[pl.ds(base, _ROWS_PER_W)], idx_v)

    def fire(rows, sem, g):
        off = g * _R
        for j in range(_K):
            pltpu.async_copy(
                table_hbm.at[idx_v.at[pl.ds(off + j * _CH, _CH)]],
                rows.at[pl.ds(j * _CH, _CH)],
                sem,
            )

    def drain(rows, sem):
        pltpu.make_async_copy(table_hbm.at[pl.ds(0, _R)], rows, sem).wait()

    fire(rows0, sem0, 0)

    def pair(t, carry):
        g0 = 2 * t
        fire(rows1, sem1, g0 + 1)
        drain(rows0, sem0)
        pltpu.sync_copy(rows0, out_hbm.at[pl.ds(base + g0 * _R, _R)])

        @pl.when(t < _G // 2 - 1)
        def _():
            fire(rows0, sem0, g0 + 2)

        drain(rows1, sem1)
        pltpu.sync_copy(rows1, out_hbm.at[pl.ds(base + (g0 + 1) * _R, _R)])
        return carry

    lax.fori_loop(0, _G // 2, pair, 0)


def kernel(token_ids, table):
    idx = token_ids.reshape(-1).astype(jnp.int32)
    out = _embed_gather(table, idx)
    return out.reshape(_BATCH, _SEQ, _D)
